# baseline (speedup 1.0000x reference)
import functools

import jax
import jax.numpy as jnp
from jax import lax
from jax.experimental import pallas as pl
from jax.experimental.pallas import tpu as pltpu
from jax.experimental.pallas import tpu_sc as plsc

NUM_GROUPS = 1000000
DIM = 64
BATCH_SIZE = 16384

VRANGE = 32768
SBW = 256
TAIL_C0 = 999936
TAIL_W = NUM_GROUPS - TAIL_C0
OUT_ROWS = BATCH_SIZE + 128
DUMMY = BATCH_SIZE
BATCH_ROWS = 128

_info = plsc.get_sparse_core_info()
_NC, _NS = _info.num_cores, _info.num_subcores
_NW = _NC * _NS
_LANES = 16

_mesh = plsc.VectorSubcoreMesh(core_axis_name="c", subcore_axis_name="s")


@functools.partial(
    pl.kernel,
    mesh=_mesh,
    out_type=jax.ShapeDtypeStruct((OUT_ROWS, 2 * DIM), jnp.float32),
    scratch_types=[
        pltpu.VMEM((BATCH_SIZE,), jnp.int32),
        pltpu.VMEM((BATCH_SIZE + _LANES,), jnp.int32),
        pltpu.VMEM((DIM, SBW), jnp.float32),
        pltpu.VMEM((DIM, TAIL_W), jnp.float32),
        pltpu.VMEM((BATCH_ROWS, 2 * DIM), jnp.float32),
        pltpu.VMEM((1, BATCH_ROWS), jnp.int32),
        pltpu.SMEM((2,), jnp.int32),
        pltpu.SemaphoreType.DMA,
    ],
    compiler_params=pltpu.CompilerParams(needs_layout_passes=False),
)
def _stream_gather(idx_hbm, tt_hbm, ttail_hbm, out_hbm, idx_all, my_hits,
                   staged, tail_v, rows_buf, b_batch, cnt_s, sem):
    wid = lax.axis_index("s") * _NC + lax.axis_index("c")
    lane = lax.iota(jnp.int32, _LANES)
    pltpu.sync_copy(idx_hbm, idx_all)

    def scan_body(i, cnt):
        v = idx_all[pl.ds(i * _LANES, _LANES)]
        m = lax.shift_right_logical(v, 15) == wid
        packed = (lax.bitwise_and(v, jnp.int32(32767)) << 14) | (
            i * _LANES + lane)
        plsc.store_compressed(my_hits.at[pl.ds(cnt, _LANES)], packed, mask=m)
        return cnt + plsc.all_reduce_population_count(m)[0]

    nh = lax.fori_loop(0, BATCH_SIZE // _LANES, scan_body, jnp.int32(0))
    cnt_s[0] = nh
    cnt_s[1] = jnp.int32(0)

    def flush():
        bc = cnt_s[1]
        for k in range(BATCH_ROWS // _LANES):
            sl = pl.ds(k * _LANES, _LANES)
            pos = k * _LANES + lane
            b_batch[0, sl] = jnp.where(pos >= bc, jnp.int32(DUMMY),
                                       b_batch[0, sl])
        pltpu.async_copy(rows_buf, out_hbm.at[b_batch.at[0]], sem).wait()
        cnt_s[1] = jnp.int32(0)

    def emit_block(c0_local, width, src):
        n_hits = cnt_s[0]

        def grp_body(g, _):
            packed = my_hits[pl.ds(g * _LANES, _LANES)]
            vi = lax.shift_right_logical(packed, 14)
            vb = lax.bitwise_and(packed, jnp.int32(16383))
            valid = (g * _LANES + lane) < n_hits
            m = valid & (vi >= c0_local) & (vi < c0_local + width)
            npc = plsc.all_reduce_population_count(m)[0]

            @pl.when(npc > 0)
            def _():
                @pl.when(cnt_s[1] > BATCH_ROWS - _LANES)
                def _():
                    flush()

                bc = cnt_s[1]
                slots = bc + plsc.cumsum(m.astype(jnp.int32)) - 1
                col = vi - c0_local
                plsc.store_scatter(b_batch, [jnp.zeros((_LANES,), jnp.int32),
                                             slots], vb, mask=m)
                for d in range(DIM):
                    dv = jnp.full((_LANES,), d, jnp.int32)
                    val = plsc.load_gather(src, [dv, col], mask=m)
                    plsc.store_scatter(rows_buf, [slots, dv], val, mask=m)
                cnt_s[1] = bc + npc

            return 0

        lax.fori_loop(0, (n_hits + _LANES - 1) // _LANES, grp_body, 0)

    base_c = wid * VRANGE
    nblk = jnp.where(wid < 30, VRANGE // SBW,
                     jnp.where(wid == 30, (TAIL_C0 - 30 * VRANGE) // SBW, 0))

    def block_body(s, _):
        c0 = pl.multiple_of(base_c + s * SBW, SBW)
        pltpu.sync_copy(tt_hbm.at[:, pl.ds(c0, SBW)], staged)
        emit_block(s * SBW, SBW, staged)
        return 0

    lax.fori_loop(0, nblk, block_body, 0)

    @pl.when(wid == _NW - 2)
    def _():
        pltpu.sync_copy(ttail_hbm, tail_v)
        emit_block(jnp.int32(TAIL_C0 - 30 * VRANGE), TAIL_W, tail_v)

    flush()


def kernel(num_group, table):
    idx = num_group.astype(jnp.int32)
    ttail = table[TAIL_C0:, :].T
    out = _stream_gather(idx, table.T, ttail)
    return out[:BATCH_SIZE, :DIM]


# speedup vs baseline: 1.1967x; 1.1967x over previous
import functools

import jax
import jax.numpy as jnp
from jax import lax
from jax.experimental import pallas as pl
from jax.experimental.pallas import tpu as pltpu
from jax.experimental.pallas import tpu_sc as plsc

NUM_GROUPS = 1000000
DIM = 64
BATCH_SIZE = 16384

VRANGE = 32768
SBW = 512
TAIL_C0 = 999936
TAIL_W = NUM_GROUPS - TAIL_C0
OUT_ROWS = BATCH_SIZE + 128
DUMMY = BATCH_SIZE
BATCH_ROWS = 128

_info = plsc.get_sparse_core_info()
_NC, _NS = _info.num_cores, _info.num_subcores
_NW = _NC * _NS
_LANES = 16

_mesh = plsc.VectorSubcoreMesh(core_axis_name="c", subcore_axis_name="s")


@functools.partial(
    pl.kernel,
    mesh=_mesh,
    out_type=jax.ShapeDtypeStruct((OUT_ROWS, 2 * DIM), jnp.float32),
    scratch_types=[
        pltpu.VMEM((BATCH_SIZE,), jnp.int32),
        pltpu.VMEM((BATCH_SIZE + _LANES,), jnp.int32),
        pltpu.VMEM((DIM, SBW), jnp.float32),
        pltpu.VMEM((DIM, TAIL_W), jnp.float32),
        pltpu.VMEM((BATCH_ROWS, 2 * DIM), jnp.float32),
        pltpu.VMEM((1, BATCH_ROWS), jnp.int32),
        pltpu.SMEM((2,), jnp.int32),
        pltpu.SemaphoreType.DMA,
    ],
    compiler_params=pltpu.CompilerParams(needs_layout_passes=False),
)
def _stream_gather(idx_hbm, tt_hbm, ttail_hbm, out_hbm, idx_all, my_hits,
                   staged, tail_v, rows_buf, b_batch, cnt_s, sem):
    wid = lax.axis_index("s") * _NC + lax.axis_index("c")
    lane = lax.iota(jnp.int32, _LANES)
    pltpu.sync_copy(idx_hbm, idx_all)

    def scan_body(i, cnt):
        v = idx_all[pl.ds(i * _LANES, _LANES)]
        m = lax.shift_right_logical(v, 15) == wid
        packed = (lax.bitwise_and(v, jnp.int32(32767)) << 14) | (
            i * _LANES + lane)
        plsc.store_compressed(my_hits.at[pl.ds(cnt, _LANES)], packed, mask=m)
        return cnt + plsc.all_reduce_population_count(m)[0]

    nh = lax.fori_loop(0, BATCH_SIZE // _LANES, scan_body, jnp.int32(0))
    cnt_s[0] = nh
    cnt_s[1] = jnp.int32(0)

    def flush():
        bc = cnt_s[1]
        for k in range(BATCH_ROWS // _LANES):
            sl = pl.ds(k * _LANES, _LANES)
            pos = k * _LANES + lane
            b_batch[0, sl] = jnp.where(pos >= bc, jnp.int32(DUMMY),
                                       b_batch[0, sl])
        pltpu.async_copy(rows_buf, out_hbm.at[b_batch.at[0]], sem).wait()
        cnt_s[1] = jnp.int32(0)

    def emit_block(c0_local, width, src):
        n_hits = cnt_s[0]

        def grp_body(g, _):
            packed = my_hits[pl.ds(g * _LANES, _LANES)]
            vi = lax.shift_right_logical(packed, 14)
            vb = lax.bitwise_and(packed, jnp.int32(16383))
            valid = (g * _LANES + lane) < n_hits
            m = valid & (vi >= c0_local) & (vi < c0_local + width)
            npc = plsc.all_reduce_population_count(m)[0]

            @pl.when(npc > 0)
            def _():
                @pl.when(cnt_s[1] > BATCH_ROWS - _LANES)
                def _():
                    flush()

                bc = cnt_s[1]
                slots = bc + plsc.cumsum(m.astype(jnp.int32)) - 1
                col = vi - c0_local
                plsc.store_scatter(b_batch, [jnp.zeros((_LANES,), jnp.int32),
                                             slots], vb, mask=m)
                for d in range(DIM):
                    dv = jnp.full((_LANES,), d, jnp.int32)
                    val = plsc.load_gather(src, [dv, col], mask=m)
                    plsc.store_scatter(rows_buf, [slots, dv], val, mask=m)
                cnt_s[1] = bc + npc

            return 0

        lax.fori_loop(0, (n_hits + _LANES - 1) // _LANES, grp_body, 0)

    base_c = wid * VRANGE
    nblk = jnp.where(wid < 30, VRANGE // SBW,
                     jnp.where(wid == 30, (TAIL_C0 - 30 * VRANGE) // SBW, 0))

    def block_body(s, _):
        c0 = pl.multiple_of(base_c + s * SBW, SBW)
        pltpu.sync_copy(tt_hbm.at[:, pl.ds(c0, SBW)], staged)
        emit_block(s * SBW, SBW, staged)
        return 0

    lax.fori_loop(0, nblk, block_body, 0)

    @pl.when(wid == _NW - 2)
    def _():
        pltpu.sync_copy(ttail_hbm, tail_v)
        emit_block(jnp.int32(TAIL_C0 - 30 * VRANGE), TAIL_W, tail_v)

    flush()


def kernel(num_group, table):
    idx = num_group.astype(jnp.int32)
    ttail = table[TAIL_C0:, :].T
    out = _stream_gather(idx, table.T, ttail)
    return out[:BATCH_SIZE, :DIM]


# speedup vs baseline: 1.2997x; 1.0861x over previous
import functools

import jax
import jax.numpy as jnp
from jax import lax
from jax.experimental import pallas as pl
from jax.experimental.pallas import tpu as pltpu
from jax.experimental.pallas import tpu_sc as plsc

NUM_GROUPS = 1000000
DIM = 64
BATCH_SIZE = 16384

VRANGE = 32768
SBW = 512
TAIL_C0 = 999936
TAIL_W = NUM_GROUPS - TAIL_C0
OUT_ROWS = BATCH_SIZE + 128
DUMMY = BATCH_SIZE
BATCH_ROWS = 128

_info = plsc.get_sparse_core_info()
_NC, _NS = _info.num_cores, _info.num_subcores
_NW = _NC * _NS
_LANES = 16

_mesh = plsc.VectorSubcoreMesh(core_axis_name="c", subcore_axis_name="s")


@functools.partial(
    pl.kernel,
    mesh=_mesh,
    out_type=jax.ShapeDtypeStruct((OUT_ROWS, 2 * DIM), jnp.float32),
    scratch_types=[
        pltpu.VMEM((BATCH_SIZE,), jnp.int32),
        pltpu.VMEM((BATCH_SIZE + _LANES,), jnp.int32),
        pltpu.VMEM((DIM, SBW), jnp.float32),
        pltpu.VMEM((DIM, SBW), jnp.float32),
        pltpu.VMEM((DIM, TAIL_W), jnp.float32),
        pltpu.VMEM((BATCH_ROWS, 2 * DIM), jnp.float32),
        pltpu.VMEM((1, BATCH_ROWS), jnp.int32),
        pltpu.SMEM((2,), jnp.int32),
        pltpu.SemaphoreType.DMA,
        pltpu.SemaphoreType.DMA,
        pltpu.SemaphoreType.DMA,
    ],
    compiler_params=pltpu.CompilerParams(needs_layout_passes=False),
)
def _stream_gather(idx_hbm, tt_hbm, ttail_hbm, out_hbm, idx_all, my_hits,
                   staged_a, staged_b, tail_v, rows_buf, b_batch, cnt_s, sem,
                   sem_a, sem_b):
    wid = lax.axis_index("s") * _NC + lax.axis_index("c")
    lane = lax.iota(jnp.int32, _LANES)
    pltpu.sync_copy(idx_hbm, idx_all)

    def scan_body(i, cnt):
        v = idx_all[pl.ds(i * _LANES, _LANES)]
        m = lax.shift_right_logical(v, 15) == wid
        packed = (lax.bitwise_and(v, jnp.int32(32767)) << 14) | (
            i * _LANES + lane)
        plsc.store_compressed(my_hits.at[pl.ds(cnt, _LANES)], packed, mask=m)
        return cnt + plsc.all_reduce_population_count(m)[0]

    nh = lax.fori_loop(0, BATCH_SIZE // _LANES, scan_body, jnp.int32(0))
    cnt_s[0] = nh
    cnt_s[1] = jnp.int32(0)

    def flush():
        bc = cnt_s[1]
        for k in range(BATCH_ROWS // _LANES):
            sl = pl.ds(k * _LANES, _LANES)
            pos = k * _LANES + lane
            b_batch[0, sl] = jnp.where(pos >= bc, jnp.int32(DUMMY),
                                       b_batch[0, sl])
        pltpu.async_copy(rows_buf, out_hbm.at[b_batch.at[0]], sem).wait()
        cnt_s[1] = jnp.int32(0)

    def emit_block(c0_local, width, src):
        n_hits = cnt_s[0]

        def grp_body(g, _):
            packed = my_hits[pl.ds(g * _LANES, _LANES)]
            vi = lax.shift_right_logical(packed, 14)
            vb = lax.bitwise_and(packed, jnp.int32(16383))
            valid = (g * _LANES + lane) < n_hits
            m = valid & (vi >= c0_local) & (vi < c0_local + width)
            npc = plsc.all_reduce_population_count(m)[0]

            @pl.when(npc > 0)
            def _():
                @pl.when(cnt_s[1] > BATCH_ROWS - _LANES)
                def _():
                    flush()

                bc = cnt_s[1]
                slots = bc + plsc.cumsum(m.astype(jnp.int32)) - 1
                col = vi - c0_local
                plsc.store_scatter(b_batch, [jnp.zeros((_LANES,), jnp.int32),
                                             slots], vb, mask=m)
                for d in range(DIM):
                    dv = jnp.full((_LANES,), d, jnp.int32)
                    val = plsc.load_gather(src, [dv, col], mask=m)
                    plsc.store_scatter(rows_buf, [slots, dv], val, mask=m)
                cnt_s[1] = bc + npc

            return 0

        lax.fori_loop(0, (n_hits + _LANES - 1) // _LANES, grp_body, 0)

    base_c = wid * VRANGE
    npair = jnp.where(wid < 30, VRANGE // (2 * SBW),
                      jnp.where(wid == 30, 16, 0))

    def stage(c0, buf, s):
        return pltpu.make_async_copy(tt_hbm.at[:, pl.ds(c0, SBW)], buf, s)

    @pl.when(npair > 0)
    def _():
        stage(pl.multiple_of(base_c, SBW), staged_a, sem_a).start()

    def pair_body(g, _):
        c0 = pl.multiple_of(base_c + 2 * g * SBW, SBW)
        stage(c0, staged_a, sem_a).wait()
        stage(c0 + SBW, staged_b, sem_b).start()
        emit_block(2 * g * SBW, SBW, staged_a)
        stage(c0 + SBW, staged_b, sem_b).wait()

        @pl.when(g + 1 < npair)
        def _():
            stage(c0 + 2 * SBW, staged_a, sem_a).start()

        emit_block((2 * g + 1) * SBW, SBW, staged_b)
        return 0

    lax.fori_loop(0, npair, pair_body, 0)

    @pl.when(wid == _NW - 2)
    def _():
        c_last = pl.multiple_of(jnp.int32(TAIL_C0 - SBW), SBW)
        pltpu.sync_copy(tt_hbm.at[:, pl.ds(c_last, SBW)], staged_a)
        emit_block(jnp.int32(TAIL_C0 - SBW - 30 * VRANGE), SBW, staged_a)
        pltpu.sync_copy(ttail_hbm, tail_v)
        emit_block(jnp.int32(TAIL_C0 - 30 * VRANGE), TAIL_W, tail_v)

    flush()


def kernel(num_group, table):
    idx = num_group.astype(jnp.int32)
    ttail = table[TAIL_C0:, :].T
    out = _stream_gather(idx, table.T, ttail)
    return out[:BATCH_SIZE, :DIM]
